# baseline (speedup 1.0000x reference)
import functools

import jax
import jax.numpy as jnp
from jax import lax
from jax.experimental import pallas as pl
from jax.experimental.pallas import tpu as pltpu
from jax.experimental.pallas import tpu_sc as plsc

NC = 2
NS = 16
EB = 128


def _make_segment_sum(n, d, nacc, nb, nc):
    zps = nacc // NS
    per_w = nb * EB

    mesh = plsc.VectorSubcoreMesh(
        core_axis_name="c", subcore_axis_name="s",
        num_cores=nc, num_subcores=NS)

    @functools.partial(
        pl.kernel,
        out_type=jax.ShapeDtypeStruct((nc, nacc, d), jnp.float32),
        mesh=mesh,
        scratch_types=[
            pltpu.VMEM_SHARED((nacc, d), jnp.float32),
            pltpu.VMEM((EB,), jnp.int32),
            pltpu.VMEM((EB,), jnp.int32),
            pltpu.VMEM((EB,), jnp.int32),
            pltpu.VMEM((EB,), jnp.int32),
            pltpu.VMEM((EB,), jnp.int32),
            pltpu.VMEM((EB,), jnp.int32),
            pltpu.VMEM((EB,), jnp.int32),
            pltpu.VMEM((EB,), jnp.int32),
            pltpu.VMEM((EB, d), jnp.float32),
            pltpu.VMEM((EB, d), jnp.float32),
            pltpu.SemaphoreType.DMA,
            pltpu.SemaphoreType.DMA,
            pltpu.SemaphoreType.DMA,
            pltpu.SemaphoreType.DMA,
            pltpu.SemaphoreType.DMA,
            pltpu.SemaphoreType.DMA,
            pltpu.SemaphoreType.DMA,
            pltpu.SemaphoreType.DMA,
        ],
    )
    def seg_sum(h_hbm, src_hbm, dst_hbm, zero_hbm, out_hbm,
                acc, sx0, sx1, sx2, sx3, dx0, dx1, dx2, dx3, rw0, rw1,
                i0, i1, i2, i3, g0, g1, a0, a1):
        c = lax.axis_index("c")
        s = lax.axis_index("s")
        wid = c * NS + s
        base = wid * per_w

        pltpu.sync_copy(zero_hbm, acc.at[pl.ds(s * zps, zps)])
        plsc.subcore_barrier()

        sx = (sx0, sx1, sx2, sx3)
        dx = (dx0, dx1, dx2, dx3)
        rw = (rw0, rw1)
        isems = (i0, i1, i2, i3)
        gsems = (g0, g1)
        asems = (a0, a1)

        def idx_descs(m, k4):
            return (pltpu.make_async_copy(
                        src_hbm.at[pl.ds(base + m * EB, EB)], sx[k4], isems[k4]),
                    pltpu.make_async_copy(
                        dst_hbm.at[pl.ds(base + m * EB, EB)], dx[k4], isems[k4]))

        def gather_desc(k4):
            return pltpu.make_async_copy(h_hbm.at[sx[k4]], rw[k4 % 2],
                                         gsems[k4 % 2])

        def scatter_desc(k4):
            return pltpu.make_async_copy(rw[k4 % 2], acc.at[dx[k4]],
                                         asems[k4 % 2])

        def start_idx(m, k4):
            for dsc in idx_descs(m, k4):
                dsc.start()

        def wait_idx(k4):
            for dsc in idx_descs(0, k4):
                dsc.wait()

        start_idx(0, 0)
        start_idx(1, 1)
        start_idx(2, 2)
        wait_idx(0)
        gather_desc(0).start()
        start_idx(3, 3)
        wait_idx(1)
        gather_desc(1).start()
        gather_desc(0).wait()
        scatter_desc(0).start(add=True)
        for k in (2, 3):
            scatter_desc(k).wait()
            start_idx(k + 2, (k + 2) % 4)
            wait_idx(k)
            gather_desc(k).start()
            gather_desc((k + 3) % 4).wait()
            scatter_desc((k + 3) % 4).start(add=True)

        def body(i, carry):
            for k in range(4):
                m = i * 4 + k
                scatter_desc(k).wait()
                @pl.when(m + 2 < nb)
                def _(m=m, k=k):
                    start_idx(m + 2, (k + 2) % 4)
                wait_idx(k)
                gather_desc(k).start()
                gather_desc((k + 3) % 4).wait()
                scatter_desc((k + 3) % 4).start(add=True)
            return carry

        lax.fori_loop(1, nb // 4, body, 0)
        gather_desc((nb - 1) % 4).wait()
        scatter_desc((nb - 1) % 4).start(add=True)
        scatter_desc((nb - 2) % 4).wait()
        scatter_desc((nb - 1) % 4).wait()
        plsc.subcore_barrier()

        pltpu.sync_copy(acc.at[pl.ds(s * zps, zps)],
                        out_hbm.at[c].at[pl.ds(s * zps, zps)])

    return seg_sum


def _linear_relu(parts, w, b, n, d, blk, nc):
    nbk = n // blk

    def body(*refs):
        p_refs, (w_ref, b_ref, o_ref) = refs[:nc], refs[nc:]
        msgs = p_refs[0][0]
        for pr in p_refs[1:]:
            msgs = msgs + pr[0]
        y = lax.dot_general(msgs, w_ref[...], (((1,), (0,)), ((), ())),
                            preferred_element_type=jnp.float32)
        o_ref[...] = jnp.maximum(y + b_ref[...], 0.0)

    in_specs = [
        pl.BlockSpec((1, blk, d), functools.partial(lambda cc, i: (cc, i, 0), cc))
        for cc in range(nc)
    ] + [
        pl.BlockSpec((d, d), lambda i: (0, 0)),
        pl.BlockSpec((1, d), lambda i: (0, 0)),
    ]
    return pl.pallas_call(
        body,
        grid=(nbk,),
        in_specs=in_specs,
        out_specs=pl.BlockSpec((blk, d), lambda i: (i, 0)),
        out_shape=jax.ShapeDtypeStruct((n, d), jnp.float32),
    )(*([parts] * nc), w, b.reshape(1, d))


def kernel(x, edge_index, W1, b1, W2, b2):
    n, d = x.shape
    e = edge_index.shape[1]

    loop = jnp.arange(n, dtype=jnp.int32)
    src = jnp.concatenate([edge_index[0].astype(jnp.int32), loop])
    dst = jnp.concatenate([edge_index[1].astype(jnp.int32), loop])

    nc = 1
    etot = e + n
    nw = nc * NS
    nb = -(-etot // (nw * EB * 4)) * 4
    epad = nw * nb * EB - etot
    src = jnp.concatenate([src, jnp.zeros((epad,), jnp.int32)])
    dst = jnp.concatenate([dst, jnp.full((epad,), n, jnp.int32)])

    nacc = -(-(n + 1) // (8 * NS)) * (8 * NS)
    zeros = jnp.zeros((nacc // NS, d), jnp.float32)

    seg = _make_segment_sum(n, d, nacc, nb, nc)

    parts1 = seg(x, src, dst, zeros)
    h1 = _linear_relu(parts1, W1, b1, n, d, 1000, nc)
    parts2 = seg(h1, src, dst, zeros)
    h2 = _linear_relu(parts2, W2, b2, n, d, 1000, nc)
    return h2
